# Initial kernel scaffold; baseline (speedup 1.0000x reference)
import jax
import jax.numpy as jnp
from jax.experimental import pallas as pl


def kernel(x, edge_index, edge_attr, u, batch, W, b):
    raise NotImplementedError("write your pallas kernel here")


# speedup vs baseline: 3.6022x; 3.6022x over previous
import functools

import jax
import jax.numpy as jnp
from jax import lax
from jax.experimental import pallas as pl
from jax.experimental.pallas import tpu as pltpu
from jax.experimental.pallas import tpu_sc as plsc

N = 10000
F = 128
B = 64
NC = 2
NS = 16
NW = NC * NS
L = 16
CH = 320
TAIL_START = 31 * CH
TAIL = N - TAIL_START


def _sc_segment_partials(x, batch):
    mesh = plsc.VectorSubcoreMesh(core_axis_name="c", subcore_axis_name="s")

    @functools.partial(
        pl.kernel,
        out_type=[
            jax.ShapeDtypeStruct((NW, B, F), jnp.float32),
            jax.ShapeDtypeStruct((NW, B, L), jnp.float32),
        ],
        mesh=mesh,
        scratch_types=[
            pltpu.VMEM((CH, F), jnp.float32),
            pltpu.VMEM((CH,), jnp.int32),
            pltpu.VMEM((B, F), jnp.float32),
            pltpu.VMEM((B, L), jnp.float32),
        ],
    )
    def sc_kernel(x_hbm, b_hbm, psum_hbm, pcnt_hbm, xv, bv, acc, cnt):
        wid = lax.axis_index("s") * NC + lax.axis_index("c")

        zeros = jnp.zeros((L,), jnp.float32)

        def zero_body(i):
            for j in range(F // L):
                acc[i, pl.ds(j * L, L)] = zeros
            cnt[i, :] = zeros

        pl.loop(0, B)(zero_body)

        ones = jnp.ones((L,), jnp.float32)

        def accumulate(start, nrows):
            pltpu.sync_copy(x_hbm.at[pl.ds(start, nrows)], xv.at[pl.ds(0, nrows)])
            pltpu.sync_copy(b_hbm.at[pl.ds(start, nrows)], bv.at[pl.ds(0, nrows)])

            def grp_body(g):
                segv = bv[pl.ds(g * L, L)]
                for k in range(L):
                    s = segv[k]
                    plsc.addupdate(cnt.at[s, :], ones)
                    for j in range(F // L):
                        v = xv[g * L + k, pl.ds(j * L, L)]
                        plsc.addupdate(acc.at[s, pl.ds(j * L, L)], v)

            pl.loop(0, nrows // L)(grp_body)

        @pl.when(wid < NW - 1)
        def _():
            accumulate(wid * CH, CH)

        @pl.when(wid == NW - 1)
        def _():
            accumulate(TAIL_START, TAIL)

        pltpu.sync_copy(acc, psum_hbm.at[wid])
        pltpu.sync_copy(cnt, pcnt_hbm.at[wid])

    return sc_kernel(x, batch)


def _tc_finish(psum, pcnt, u, W, b2):
    def tc_body(ps_ref, pc_ref, u_ref, w_ref, b_ref, out_ref):
        sums = jnp.sum(ps_ref[...], axis=0)
        counts = jnp.sum(pc_ref[...], axis=0)[:, :1]
        x_agg = sums / jnp.maximum(counts, 1.0)
        w = w_ref[...]
        out = (
            jnp.dot(x_agg, w[:F], preferred_element_type=jnp.float32)
            + jnp.dot(u_ref[...], w[F:], preferred_element_type=jnp.float32)
            + b_ref[...]
        )
        out_ref[...] = out

    return pl.pallas_call(
        tc_body,
        out_shape=jax.ShapeDtypeStruct((B, B), jnp.float32),
    )(psum, pcnt, u, W, b2)


def kernel(x, edge_index, edge_attr, u, batch, W, b):
    psum, pcnt = _sc_segment_partials(x, batch)
    return _tc_finish(psum, pcnt, u, W, b.reshape(1, B))


# speedup vs baseline: 3.6911x; 1.0247x over previous
import functools

import jax
import jax.numpy as jnp
from jax import lax
from jax.experimental import pallas as pl
from jax.experimental.pallas import tpu as pltpu
from jax.experimental.pallas import tpu_sc as plsc

N = 10000
F = 128
B = 64
NC = 2
NS = 16
NW = NC * NS
L = 16
CH = 320
TAIL_START = 31 * CH
TAIL = N - TAIL_START
HALF = CH // 2
NJ = F // L


def _sc_segment_partials(x, batch):
    mesh = plsc.VectorSubcoreMesh(core_axis_name="c", subcore_axis_name="s")

    @functools.partial(
        pl.kernel,
        out_type=[
            jax.ShapeDtypeStruct((NW, B * F), jnp.float32),
            jax.ShapeDtypeStruct((NW, B * L), jnp.float32),
        ],
        mesh=mesh,
        scratch_types=[
            pltpu.VMEM((HALF, F), jnp.float32),
            pltpu.VMEM((HALF, F), jnp.float32),
            pltpu.VMEM((CH,), jnp.int32),
            pltpu.VMEM((B * F,), jnp.float32),
            pltpu.VMEM((B * L,), jnp.float32),
            pltpu.SemaphoreType.DMA,
            pltpu.SemaphoreType.DMA,
        ],
    )
    def sc_kernel(x_hbm, b_hbm, psum_hbm, pcnt_hbm, xa, xb, bv, acc, cnt,
                  sa, sb):
        wid = lax.axis_index("s") * NC + lax.axis_index("c")

        zeros = jnp.zeros((L,), jnp.float32)
        ones_v = jnp.ones((L,), jnp.float32)

        def flush(seg, accv, cntf):
            plsc.addupdate(cnt.at[pl.ds(seg * L, L)], cntf)
            for j in range(NJ):
                plsc.addupdate(acc.at[pl.ds(seg * F + j * L, L)], accv[j])

        def chunk_groups(xv, goff, ngrp, carry):
            def grp_body(g, c):
                cur, cntf, accv = c
                segv = bv[pl.ds((goff + g) * L, L)]
                for k in range(L):
                    s = segv[k]
                    is_new = s != cur

                    @pl.when(is_new)
                    def _():
                        flush(cur, accv, cntf)

                    keep = jnp.where(is_new, 0.0, 1.0)
                    row = [xv[g * L + k, pl.ds(j * L, L)] for j in range(NJ)]
                    accv = [accv[j] * keep + row[j] for j in range(NJ)]
                    cntf = cntf * keep + ones_v
                    cur = s
                return (cur, cntf, accv)

            return pl.loop(0, ngrp, init_carry=carry)(grp_body)

        def zero_acc():
            def zero_body(i):
                for j in range(NJ):
                    acc[pl.ds(i * F + j * L, L)] = zeros
                cnt[pl.ds(i * L, L)] = zeros

            pl.loop(0, B)(zero_body)

        def init_carry():
            s0 = bv[pl.ds(0, L)][0]
            return (s0, zeros, [zeros for _ in range(NJ)])

        @pl.when(wid < NW - 1)
        def _():
            start = wid * CH
            c0 = pltpu.async_copy(x_hbm.at[pl.ds(start, HALF)], xa, sa)
            c1 = pltpu.async_copy(x_hbm.at[pl.ds(start + HALF, HALF)], xb, sb)
            pltpu.sync_copy(b_hbm.at[pl.ds(start, CH)], bv)
            zero_acc()
            carry = init_carry()
            c0.wait()
            carry = chunk_groups(xa, 0, HALF // L, carry)
            c1.wait()
            carry = chunk_groups(xb, HALF // L, HALF // L, carry)
            flush(carry[0], carry[2], carry[1])

        @pl.when(wid == NW - 1)
        def _():
            c0 = pltpu.async_copy(x_hbm.at[pl.ds(TAIL_START, TAIL)],
                                  xa.at[pl.ds(0, TAIL)], sa)
            pltpu.sync_copy(b_hbm.at[pl.ds(TAIL_START, TAIL)],
                            bv.at[pl.ds(0, TAIL)])
            zero_acc()
            carry = init_carry()
            c0.wait()
            carry = chunk_groups(xa, 0, TAIL // L, carry)
            flush(carry[0], carry[2], carry[1])

        pltpu.sync_copy(acc, psum_hbm.at[wid])
        pltpu.sync_copy(cnt, pcnt_hbm.at[wid])

    return sc_kernel(x, batch)


def _tc_finish(psum, pcnt, u, W, b2):
    def tc_body(ps_ref, pc_ref, u_ref, w_ref, b_ref, out_ref):
        sums = jnp.sum(ps_ref[...], axis=0)
        counts = jnp.sum(pc_ref[...], axis=0)[:, :1]
        x_agg = sums / jnp.maximum(counts, 1.0)
        w = w_ref[...]
        out = (
            jnp.dot(x_agg, w[:F], preferred_element_type=jnp.float32)
            + jnp.dot(u_ref[...], w[F:], preferred_element_type=jnp.float32)
            + b_ref[...]
        )
        out_ref[...] = out

    return pl.pallas_call(
        tc_body,
        out_shape=jax.ShapeDtypeStruct((B, B), jnp.float32),
    )(psum, pcnt, u, W, b2)


def kernel(x, edge_index, edge_attr, u, batch, W, b):
    psum, pcnt = _sc_segment_partials(x, batch)
    return _tc_finish(psum.reshape(NW, B, F), pcnt.reshape(NW, B, L),
                      u, W, b.reshape(1, B))
